# baseline (speedup 1.0000x reference)
import jax
import jax.numpy as jnp
from jax import lax
from jax.experimental import pallas as pl
from jax.experimental.pallas import tpu as pltpu
from jax.experimental.pallas import tpu_sc as plsc

NW = 32
PAIRS = 98304
PER_W = PAIRS // NW
CHUNK = 128
NCH = PER_W // CHUNK
EMB = 64
N_POS = 16384


def _sc_dots_body(ii_hbm, oi_hbm, in_hbm, out_hbm, part_hbm,
                  ii_v, oi_v, rin_v, rout_v, part_v, sem_i, sem_o):
    wid = lax.axis_index("s") * 2 + lax.axis_index("c")
    pltpu.sync_copy(ii_hbm.at[wid], ii_v)
    pltpu.sync_copy(oi_hbm.at[wid], oi_v)

    def chunk_body(c, carry):
        cp_i = pltpu.async_copy(in_hbm.at[ii_v.at[c]], rin_v, sem_i)
        cp_o = pltpu.async_copy(out_hbm.at[oi_v.at[c]], rout_v, sem_o)
        cp_i.wait()
        cp_o.wait()

        def pair_body(p, carry2):
            acc = rin_v[p, pl.ds(0, 16)] * rout_v[p, pl.ds(0, 16)]
            for k in range(1, EMB // 16):
                acc = acc + rin_v[p, pl.ds(16 * k, 16)] * rout_v[p, pl.ds(16 * k, 16)]
            part_v[c * CHUNK + p, :] = acc
            return carry2

        lax.fori_loop(0, CHUNK, pair_body, 0)
        return carry

    lax.fori_loop(0, NCH, chunk_body, 0)
    pltpu.sync_copy(part_v, part_hbm.at[pl.ds(wid * PER_W, PER_W)])


_sc_dots = pl.kernel(
    _sc_dots_body,
    mesh=plsc.VectorSubcoreMesh(core_axis_name="c", subcore_axis_name="s"),
    compiler_params=pltpu.CompilerParams(use_tc_tiling_on_sc=False),
    out_type=jax.ShapeDtypeStruct((PAIRS, 16), jnp.float32),
    scratch_types=[
        pltpu.VMEM((NCH, CHUNK), jnp.int32),
        pltpu.VMEM((NCH, CHUNK), jnp.int32),
        pltpu.VMEM((CHUNK, EMB), jnp.float32),
        pltpu.VMEM((CHUNK, EMB), jnp.float32),
        pltpu.VMEM((PER_W, 16), jnp.float32),
        pltpu.SemaphoreType.DMA,
        pltpu.SemaphoreType.DMA,
    ],
)


def _tc_loss_body(part_ref, out_ref):
    x = part_ref[...]
    col = lax.broadcasted_iota(jnp.int32, (256, 16), 0)
    grp = lax.broadcasted_iota(jnp.int32, (256, 16), 1)
    g = jnp.where(col // 16 == grp, 1.0, 0.0)
    s = jnp.dot(x, g, preferred_element_type=jnp.float32)
    row = lax.broadcasted_iota(jnp.int32, s.shape, 0)
    sign = jnp.where(row < N_POS // 16, 1.0, -1.0)
    y = sign * s
    ls = jnp.minimum(y, 0.0) - jnp.log(1.0 + jnp.exp(-jnp.abs(y)))
    out_ref[...] = jnp.full((1, 1), -jnp.sum(ls), jnp.float32)


_tc_loss = pl.pallas_call(
    _tc_loss_body,
    out_shape=jax.ShapeDtypeStruct((1, 1), jnp.float32),
)


def kernel(pos_in, pos_out, neg_in, neg_out, in_emb, out_emb):
    ii = jnp.concatenate([pos_in, neg_in]).astype(jnp.int32).reshape(NW, NCH, CHUNK)
    oi = jnp.concatenate([pos_out, neg_out]).astype(jnp.int32).reshape(NW, NCH, CHUNK)
    part = _sc_dots(ii, oi, in_emb, out_emb)
    loss = _tc_loss(part.reshape(PAIRS // 16, 256))
    return loss[0, 0]


# speedup vs baseline: 1.4512x; 1.4512x over previous
import jax
import jax.numpy as jnp
from jax import lax
from jax.experimental import pallas as pl
from jax.experimental.pallas import tpu as pltpu
from jax.experimental.pallas import tpu_sc as plsc

NW = 32
PAIRS = 98304
PER_W = PAIRS // NW
CHUNK = 256
NCH = PER_W // CHUNK
EMB = 64
N_POS = 16384


def _sc_dots_body(ii_hbm, oi_hbm, in_hbm, out_hbm, part_hbm,
                  ii_v, oi_v, rin_v, rout_v, part_v, sem_i, sem_o):
    wid = lax.axis_index("s") * 2 + lax.axis_index("c")
    pltpu.sync_copy(ii_hbm.at[wid], ii_v)
    pltpu.sync_copy(oi_hbm.at[wid], oi_v)

    def chunk_body(c, carry):
        def fire_body(g, carry2):
            vi = ii_v[pl.ds(c * CHUNK + g * 16, 16)]
            vo = oi_v[pl.ds(c * CHUNK + g * 16, 16)]
            for l in range(16):
                pltpu.async_copy(in_hbm.at[vi[l]], rin_v.at[g * 16 + l], sem_i)
                pltpu.async_copy(out_hbm.at[vo[l]], rout_v.at[g * 16 + l], sem_o)
            return carry2

        lax.fori_loop(0, CHUNK // 16, fire_body, 0)
        pltpu.make_async_copy(in_hbm.at[pl.ds(0, CHUNK)], rin_v, sem_i).wait()
        pltpu.make_async_copy(out_hbm.at[pl.ds(0, CHUNK)], rout_v, sem_o).wait()

        def pair_body(p, carry2):
            acc = rin_v[p, pl.ds(0, 16)] * rout_v[p, pl.ds(0, 16)]
            for k in range(1, EMB // 16):
                acc = acc + rin_v[p, pl.ds(16 * k, 16)] * rout_v[p, pl.ds(16 * k, 16)]
            part_v[p, :] = acc
            return carry2

        lax.fori_loop(0, CHUNK, pair_body, 0)
        pltpu.sync_copy(part_v, part_hbm.at[pl.ds(wid * PER_W + c * CHUNK, CHUNK)])
        return carry

    lax.fori_loop(0, NCH, chunk_body, 0)


_sc_dots = pl.kernel(
    _sc_dots_body,
    mesh=plsc.VectorSubcoreMesh(core_axis_name="c", subcore_axis_name="s"),
    out_type=jax.ShapeDtypeStruct((PAIRS, 16), jnp.float32),
    scratch_types=[
        pltpu.VMEM((PER_W,), jnp.int32),
        pltpu.VMEM((PER_W,), jnp.int32),
        pltpu.VMEM((CHUNK, EMB), jnp.float32),
        pltpu.VMEM((CHUNK, EMB), jnp.float32),
        pltpu.VMEM((CHUNK, 16), jnp.float32),
        pltpu.SemaphoreType.DMA,
        pltpu.SemaphoreType.DMA,
    ],
)


def _tc_loss_body(part_ref, out_ref):
    x = part_ref[...]
    col = lax.broadcasted_iota(jnp.int32, (256, 16), 0)
    grp = lax.broadcasted_iota(jnp.int32, (256, 16), 1)
    g = jnp.where(col // 16 == grp, 1.0, 0.0)
    s = jnp.dot(x, g, preferred_element_type=jnp.float32)
    row = lax.broadcasted_iota(jnp.int32, s.shape, 0)
    sign = jnp.where(row < N_POS // 16, 1.0, -1.0)
    y = sign * s
    ls = jnp.minimum(y, 0.0) - jnp.log(1.0 + jnp.exp(-jnp.abs(y)))
    out_ref[...] = jnp.full((1, 1), -jnp.sum(ls), jnp.float32)


_tc_loss = pl.pallas_call(
    _tc_loss_body,
    out_shape=jax.ShapeDtypeStruct((1, 1), jnp.float32),
)


def kernel(pos_in, pos_out, neg_in, neg_out, in_emb, out_emb):
    ii = jnp.concatenate([pos_in, neg_in]).astype(jnp.int32).reshape(NW, PER_W)
    oi = jnp.concatenate([pos_out, neg_out]).astype(jnp.int32).reshape(NW, PER_W)
    part = _sc_dots(ii, oi, in_emb, out_emb)
    loss = _tc_loss(part.reshape(PAIRS // 16, 256))
    return loss[0, 0]
